# baseline (speedup 1.0000x reference)
import jax
import jax.numpy as jnp
from jax.experimental import pallas as pl
from jax.experimental.pallas import tpu as pltpu

_QK_DIM = 96
_TOPK = 4
_SCALE = _QK_DIM ** (-0.5)


def _route_kernel(n_batch, g_ref, wq_ref, bq_ref, wk_ref, bk_ref,
                  w_ref, i_ref, x_scr):
    s = pl.program_id(0)
    wb = jax.lax.rem(s, 2)
    rb = 1 - wb

    g = g_ref[0]
    qh = jax.lax.dot_general(
        g, wq_ref[...], (((1,), (1,)), ((), ())),
        preferred_element_type=jnp.float32) + bq_ref[...]
    kh = jax.lax.dot_general(
        g, wk_ref[...], (((1,), (1,)), ((), ())),
        preferred_element_type=jnp.float32) + bk_ref[...]
    x_scr[wb] = jax.lax.dot_general(
        qh * _SCALE, kh, (((1,), (1,)), ((), ())),
        preferred_element_type=jnp.float32)

    x0 = x_scr[rb]
    n = x0.shape[1]

    idx_w = jnp.concatenate(
        [jax.lax.broadcasted_iota(jnp.int32, (n, 1), 0).astype(jnp.float32),
         jnp.ones((n, 1), jnp.float32)], axis=1)
    x = x0
    ds, sums, cnts = [], [], []
    for j in range(_TOPK):
        d = jnp.max(x, axis=1, keepdims=True)
        hit = x == d
        hitf = jnp.where(hit, 1.0, 0.0)
        sc = jax.lax.dot_general(
            hitf, idx_w, (((1,), (0,)), ((), ())),
            preferred_element_type=jnp.float32)
        ds.append(d)
        sums.append(sc[:, 0:1])
        cnts.append(sc[:, 1:2])
        if j + 1 < _TOPK:
            x = jnp.where(hit, -jnp.inf, x)
    cnt = jnp.concatenate(cnts, axis=1)
    need_fix = jnp.any(cnt != 1.0)

    @pl.when(jnp.logical_not(need_fix))
    def _fast():
        v = jnp.concatenate(ds, axis=1)
        w = jnp.exp(v - ds[0])
        w_ref[0] = w / jnp.sum(w, axis=1, keepdims=True)
        i_ref[0] = jnp.concatenate(sums, axis=1).astype(jnp.int32)

    @pl.when(need_fix)
    def _exact():
        iota = jax.lax.broadcasted_iota(jnp.int32, x0.shape, 1)
        y = x0
        vals, idxs = [], []
        for j in range(_TOPK):
            m = jnp.max(y, axis=1, keepdims=True)
            idx = jnp.min(jnp.where(y == m, iota, n), axis=1, keepdims=True)
            vals.append(m)
            idxs.append(idx)
            if j + 1 < _TOPK:
                y = jnp.where(iota == idx, -jnp.inf, y)
        v = jnp.concatenate(vals, axis=1)
        w = jnp.exp(v - vals[0])
        w_ref[0] = w / jnp.sum(w, axis=1, keepdims=True)
        i_ref[0] = jnp.concatenate(idxs, axis=1)


@jax.jit
def kernel(g_win, Wq, bq, Wk, bk):
    B, N, D = g_win.shape
    import functools
    body = functools.partial(_route_kernel, B)
    out = pl.pallas_call(
        body,
        grid=(B + 1,),
        in_specs=[
            pl.BlockSpec((1, N, D), lambda b: (jnp.minimum(b, 31), 0, 0)),
            pl.BlockSpec((D, D), lambda b: (0, 0)),
            pl.BlockSpec((1, D), lambda b: (0, 0)),
            pl.BlockSpec((D, D), lambda b: (0, 0)),
            pl.BlockSpec((1, D), lambda b: (0, 0)),
        ],
        out_specs=[
            pl.BlockSpec((1, N, _TOPK),
                         lambda b: (jnp.maximum(b - 1, 0), 0, 0)),
            pl.BlockSpec((1, N, _TOPK),
                         lambda b: (jnp.maximum(b - 1, 0), 0, 0)),
        ],
        out_shape=[
            jax.ShapeDtypeStruct((B, N, _TOPK), jnp.float32),
            jax.ShapeDtypeStruct((B, N, _TOPK), jnp.int32),
        ],
        scratch_shapes=[pltpu.VMEM((2, N, N), jnp.float32)],
    )(g_win, Wq, bq.reshape(1, D), Wk, bk.reshape(1, D))
    return out[0], out[1]


# speedup vs baseline: 1.1947x; 1.1947x over previous
import jax
import jax.numpy as jnp
from jax.experimental import pallas as pl

_QK_DIM = 96
_TOPK = 4
_BPS = 2
_SCALE = _QK_DIM ** (-0.5)


def _route_kernel(g_ref, wq_ref, bq_ref, wk_ref, bk_ref, w_ref, i_ref):
    x0s = []
    for t in range(_BPS):
        g = g_ref[t]
        qh = jax.lax.dot_general(
            g, wq_ref[...], (((1,), (1,)), ((), ())),
            preferred_element_type=jnp.float32) + bq_ref[...]
        kh = jax.lax.dot_general(
            g, wk_ref[...], (((1,), (1,)), ((), ())),
            preferred_element_type=jnp.float32) + bk_ref[...]
        x0s.append(jax.lax.dot_general(
            qh * _SCALE, kh, (((1,), (1,)), ((), ())),
            preferred_element_type=jnp.float32))

    n = x0s[0].shape[1]
    idx_w = jnp.concatenate(
        [jax.lax.broadcasted_iota(jnp.int32, (n, 1), 0).astype(jnp.float32),
         jnp.ones((n, 1), jnp.float32)], axis=1)

    xs = list(x0s)
    ds = [[] for _ in range(_BPS)]
    sums = [[] for _ in range(_BPS)]
    cnts = [[] for _ in range(_BPS)]
    for j in range(_TOPK):
        hits = [None] * _BPS
        for t in range(_BPS):
            d = jnp.max(xs[t], axis=1, keepdims=True)
            hits[t] = xs[t] == d
            ds[t].append(d)
        for t in range(_BPS):
            hitf = jnp.where(hits[t], 1.0, 0.0)
            sc = jax.lax.dot_general(
                hitf, idx_w, (((1,), (0,)), ((), ())),
                preferred_element_type=jnp.float32)
            sums[t].append(sc[:, 0:1])
            cnts[t].append(sc[:, 1:2])
        if j + 1 < _TOPK:
            for t in range(_BPS):
                xs[t] = jnp.where(hits[t], -jnp.inf, xs[t])

    for t in range(_BPS):
        cnt = jnp.concatenate(cnts[t], axis=1)
        need_fix = jnp.any(cnt != 1.0)

        @pl.when(jnp.logical_not(need_fix))
        def _fast(t=t):
            v = jnp.concatenate(ds[t], axis=1)
            w = jnp.exp(v - ds[t][0])
            w_ref[t] = w / jnp.sum(w, axis=1, keepdims=True)
            i_ref[t] = jnp.concatenate(sums[t], axis=1).astype(jnp.int32)

        @pl.when(need_fix)
        def _exact(t=t):
            iota = jax.lax.broadcasted_iota(jnp.int32, (n, n), 1)
            y = x0s[t]
            vals, idxs = [], []
            for j in range(_TOPK):
                m = jnp.max(y, axis=1, keepdims=True)
                idx = jnp.min(jnp.where(y == m, iota, n),
                              axis=1, keepdims=True)
                vals.append(m)
                idxs.append(idx)
                if j + 1 < _TOPK:
                    y = jnp.where(iota == idx, -jnp.inf, y)
            v = jnp.concatenate(vals, axis=1)
            w = jnp.exp(v - vals[0])
            w_ref[t] = w / jnp.sum(w, axis=1, keepdims=True)
            i_ref[t] = jnp.concatenate(idxs, axis=1)


@jax.jit
def kernel(g_win, Wq, bq, Wk, bk):
    B, N, D = g_win.shape
    out = pl.pallas_call(
        _route_kernel,
        grid=(B // _BPS,),
        in_specs=[
            pl.BlockSpec((_BPS, N, D), lambda b: (b, 0, 0)),
            pl.BlockSpec((D, D), lambda b: (0, 0)),
            pl.BlockSpec((1, D), lambda b: (0, 0)),
            pl.BlockSpec((D, D), lambda b: (0, 0)),
            pl.BlockSpec((1, D), lambda b: (0, 0)),
        ],
        out_specs=[
            pl.BlockSpec((_BPS, N, _TOPK), lambda b: (b, 0, 0)),
            pl.BlockSpec((_BPS, N, _TOPK), lambda b: (b, 0, 0)),
        ],
        out_shape=[
            jax.ShapeDtypeStruct((B, N, _TOPK), jnp.float32),
            jax.ShapeDtypeStruct((B, N, _TOPK), jnp.int32),
        ],
    )(g_win, Wq, bq.reshape(1, D), Wk, bk.reshape(1, D))
    return out[0], out[1]
